# baseline (speedup 1.0000x reference)
import jax
import jax.numpy as jnp
from jax import lax
from jax.experimental import pallas as pl
from jax.experimental.pallas import tpu as pltpu
from jax.experimental.pallas import tpu_sc as plsc

N = 10000
E = 320000
D = 128
NC = 2
NS = 16
NW = NC * NS
LANES = 16
K = 128
NCHUNK = 79
EPAD = NW * NCHUNK * K
NPAD = 10112
RPS = NPAD // NS

_mesh = plsc.VectorSubcoreMesh(core_axis_name="c", subcore_axis_name="s")


def _hist_body(cols_hbm, zeros_hbm, ones_hbm, out_hbm, colv, onesv, sdeg, sem):
    c = lax.axis_index("c")
    s = lax.axis_index("s")
    wid = c * NS + s
    pltpu.sync_copy(zeros_hbm.at[pl.ds(s * RPS, RPS)],
                    sdeg.at[pl.ds(s * RPS, RPS)])
    pltpu.sync_copy(cols_hbm.at[wid], colv)
    pltpu.sync_copy(ones_hbm, onesv)
    plsc.subcore_barrier()

    @pl.loop(0, NCHUNK)
    def _(j):
        pltpu.sync_copy(onesv, sdeg.at[colv.at[j]], add=True)

    plsc.subcore_barrier()
    pltpu.sync_copy(sdeg.at[pl.ds(s * RPS, RPS)],
                    out_hbm.at[c, pl.ds(s * RPS, RPS)])


@jax.jit
def _hist(cols, zeros128, ones128):
    kern = pl.kernel(
        _hist_body,
        out_type=jax.ShapeDtypeStruct((NC, NPAD, D), jnp.float32),
        mesh=_mesh,
        scratch_types=[
            pltpu.VMEM((NCHUNK, K), jnp.int32),
            pltpu.VMEM((K, D), jnp.float32),
            pltpu.VMEM_SHARED((NPAD, D), jnp.float32),
            pltpu.SemaphoreType.DMA,
        ],
    )
    return kern(cols, zeros128, ones128)


def _spmm_body(hp_hbm, rows_hbm, cols_hbm, zeros_hbm, out_hbm,
               rowv, colv, rbuf, sacc, sem):
    c = lax.axis_index("c")
    s = lax.axis_index("s")
    wid = c * NS + s
    pltpu.sync_copy(zeros_hbm.at[pl.ds(s * RPS, RPS)],
                    sacc.at[pl.ds(s * RPS, RPS)])
    pltpu.sync_copy(rows_hbm.at[wid], rowv)
    pltpu.sync_copy(cols_hbm.at[wid], colv)
    plsc.subcore_barrier()

    @pl.loop(0, NCHUNK)
    def _(j):
        pltpu.async_copy(hp_hbm.at[rowv.at[j]], rbuf, sem).wait()
        pltpu.sync_copy(rbuf, sacc.at[colv.at[j]], add=True)

    plsc.subcore_barrier()
    pltpu.sync_copy(sacc.at[pl.ds(s * RPS, RPS)],
                    out_hbm.at[c, pl.ds(s * RPS, RPS)])


@jax.jit
def _spmm(hp_pad, rows, cols, zeros128):
    kern = pl.kernel(
        _spmm_body,
        out_type=jax.ShapeDtypeStruct((NC, NPAD, D), jnp.float32),
        mesh=_mesh,
        scratch_types=[
            pltpu.VMEM((NCHUNK, K), jnp.int32),
            pltpu.VMEM((NCHUNK, K), jnp.int32),
            pltpu.VMEM((K, D), jnp.float32),
            pltpu.VMEM_SHARED((NPAD, D), jnp.float32),
            pltpu.SemaphoreType.DMA,
        ],
    )
    return kern(hp_pad, rows, cols, zeros128)


def _tc_mm1_body(x_ref, w_ref, hw_ref):
    hw_ref[...] = jnp.dot(x_ref[...], w_ref[...],
                          preferred_element_type=jnp.float32)


def _tc_first_body(deg_ref, hw_ref, dis_ref, hp_ref):
    deg = deg_ref[0, :N, 0] + deg_ref[1, :N, 0] + 1.0
    dis = lax.rsqrt(deg)
    dis_ref[...] = dis
    hp_ref[...] = hw_ref[...] * dis[:, None]


def _tc_mid_body(s_ref, hp_ref, dis_ref, g_ref, be_ref, w_ref, out_ref):
    dis = dis_ref[...]
    pre = (s_ref[0, :N, :] + s_ref[1, :N, :] + hp_ref[...]) * dis[:, None]
    mu = jnp.mean(pre, axis=0)
    var = jnp.mean((pre - mu[None, :]) ** 2, axis=0)
    z = g_ref[...][None, :] * (pre - mu[None, :]) * lax.rsqrt(var + 1e-5)[None, :] \
        + be_ref[...][None, :]
    r = jnp.maximum(z, 0.0)
    h = jnp.dot(r, w_ref[...], preferred_element_type=jnp.float32)
    out_ref[...] = h * dis[:, None]


def _tc_final_body(s_ref, hp_ref, dis_ref, b_ref, out_ref):
    pre = (s_ref[0, :N, :] + s_ref[1, :N, :] + hp_ref[...]) * dis_ref[...][:, None]
    out_ref[...] = pre + b_ref[...][None, :]


@jax.jit
def _tc_mm1(x, w):
    return pl.pallas_call(
        _tc_mm1_body,
        out_shape=jax.ShapeDtypeStruct((N, D), jnp.float32),
    )(x, w)


@jax.jit
def _tc_first(deg, hw):
    return pl.pallas_call(
        _tc_first_body,
        out_shape=(jax.ShapeDtypeStruct((N,), jnp.float32),
                   jax.ShapeDtypeStruct((N, D), jnp.float32)),
    )(deg, hw)


@jax.jit
def _tc_mid(s_part, hp, dis, g, be, w):
    return pl.pallas_call(
        _tc_mid_body,
        out_shape=jax.ShapeDtypeStruct((N, D), jnp.float32),
    )(s_part, hp, dis, g, be, w)


@jax.jit
def _tc_final(s_part, hp, dis, b):
    return pl.pallas_call(
        _tc_final_body,
        out_shape=jax.ShapeDtypeStruct((N, D), jnp.float32),
    )(s_part, hp, dis, b)


def kernel(x, edge_index, W1, b1, g1, be1, W2, b2, g2, be2, W3, b3):
    pad = EPAD - E
    rows = jnp.concatenate(
        [edge_index[0], jnp.full((pad,), N, jnp.int32)]).reshape(NW, NCHUNK, K)
    cols = jnp.concatenate(
        [edge_index[1], jnp.full((pad,), N, jnp.int32)]).reshape(NW, NCHUNK, K)
    zeros128 = jnp.zeros((NPAD, D), jnp.float32)
    ones128 = jnp.ones((K, D), jnp.float32)

    deg = _hist(cols, zeros128, ones128)
    hw1 = _tc_mm1(x, W1)
    dis, h1p = _tc_first(deg, hw1)
    s1 = _spmm(jnp.pad(h1p, ((0, NPAD - N), (0, 0))), rows, cols, zeros128)
    h2p = _tc_mid(s1, h1p, dis, g1, be1, W2)
    s2 = _spmm(jnp.pad(h2p, ((0, NPAD - N), (0, 0))), rows, cols, zeros128)
    h3p = _tc_mid(s2, h2p, dis, g2, be2, W3)
    s3 = _spmm(jnp.pad(h3p, ((0, NPAD - N), (0, 0))), rows, cols, zeros128)
    return _tc_final(s3, h3p, dis, b3)


# speedup vs baseline: 1.1070x; 1.1070x over previous
import jax
import jax.numpy as jnp
from jax import lax
from jax.experimental import pallas as pl
from jax.experimental.pallas import tpu as pltpu
from jax.experimental.pallas import tpu_sc as plsc

N = 10000
E = 320000
D = 128
NC = 2
NS = 16
NW = NC * NS
LANES = 16
K = 128
NCHUNK = 79
EPAD = NW * NCHUNK * K
NPAD = 10112
RPS = NPAD // NS

_mesh = plsc.VectorSubcoreMesh(core_axis_name="c", subcore_axis_name="s")


def _hist_body(cols_hbm, zeros_hbm, ones_hbm, out_hbm, colv, onesv, sdeg, sem):
    c = lax.axis_index("c")
    s = lax.axis_index("s")
    wid = c * NS + s
    pltpu.sync_copy(zeros_hbm.at[pl.ds(s * RPS, RPS)],
                    sdeg.at[pl.ds(s * RPS, RPS)])
    pltpu.sync_copy(cols_hbm.at[wid], colv)
    pltpu.sync_copy(ones_hbm, onesv)
    plsc.subcore_barrier()

    @pl.loop(0, NCHUNK)
    def _(j):
        pltpu.sync_copy(onesv, sdeg.at[colv.at[j]], add=True)

    plsc.subcore_barrier()
    pltpu.sync_copy(sdeg.at[pl.ds(s * RPS, RPS)],
                    out_hbm.at[c, pl.ds(s * RPS, RPS)])


@jax.jit
def _hist(cols, zeros128, ones128):
    kern = pl.kernel(
        _hist_body,
        out_type=jax.ShapeDtypeStruct((NC, NPAD, D), jnp.float32),
        mesh=_mesh,
        scratch_types=[
            pltpu.VMEM((NCHUNK, K), jnp.int32),
            pltpu.VMEM((K, D), jnp.float32),
            pltpu.VMEM_SHARED((NPAD, D), jnp.float32),
            pltpu.SemaphoreType.DMA,
        ],
    )
    return kern(cols, zeros128, ones128)


def _spmm_body(hp_hbm, rows_hbm, cols_hbm, zeros_hbm, out_hbm,
               rowv, colv, rbuf, sacc, sem):
    c = lax.axis_index("c")
    s = lax.axis_index("s")
    wid = c * NS + s
    pltpu.sync_copy(zeros_hbm.at[pl.ds(s * RPS, RPS)],
                    sacc.at[pl.ds(s * RPS, RPS)])
    pltpu.sync_copy(rows_hbm.at[wid], rowv)
    pltpu.sync_copy(cols_hbm.at[wid], colv)
    plsc.subcore_barrier()

    @pl.loop(0, NCHUNK)
    def _(j):
        pltpu.async_copy(hp_hbm.at[rowv.at[j]], rbuf, sem).wait()
        pltpu.sync_copy(rbuf, sacc.at[colv.at[j]], add=True)

    plsc.subcore_barrier()
    pltpu.sync_copy(sacc.at[pl.ds(s * RPS, RPS)],
                    out_hbm.at[c, pl.ds(s * RPS, RPS)])


@jax.jit
def _spmm(hp_pad, rows, cols, zeros128):
    kern = pl.kernel(
        _spmm_body,
        out_type=jax.ShapeDtypeStruct((NC, NPAD, D), jnp.float32),
        mesh=_mesh,
        scratch_types=[
            pltpu.VMEM((NCHUNK, K), jnp.int32),
            pltpu.VMEM((NCHUNK, K), jnp.int32),
            pltpu.VMEM((K, D), jnp.float32),
            pltpu.VMEM_SHARED((NPAD, D), jnp.float32),
            pltpu.SemaphoreType.DMA,
        ],
    )
    return kern(hp_pad, rows, cols, zeros128)


def _tc_first_body(deg_ref, x_ref, w_ref, dis_ref, hp_ref):
    deg = deg_ref[0, :N, 0] + deg_ref[1, :N, 0] + 1.0
    dis = lax.rsqrt(deg)
    dis_ref[...] = dis
    h = jnp.dot(x_ref[...], w_ref[...], preferred_element_type=jnp.float32)
    hp_ref[...] = h * dis[:, None]


def _tc_mid_body(s_ref, hp_ref, dis_ref, g_ref, be_ref, w_ref, out_ref):
    dis = dis_ref[...]
    pre = (s_ref[0, :N, :] + s_ref[1, :N, :] + hp_ref[...]) * dis[:, None]
    mu = jnp.mean(pre, axis=0)
    var = jnp.mean((pre - mu[None, :]) ** 2, axis=0)
    z = g_ref[...][None, :] * (pre - mu[None, :]) * lax.rsqrt(var + 1e-5)[None, :] \
        + be_ref[...][None, :]
    r = jnp.maximum(z, 0.0)
    h = jnp.dot(r, w_ref[...], preferred_element_type=jnp.float32)
    out_ref[...] = h * dis[:, None]


def _tc_final_body(s_ref, hp_ref, dis_ref, b_ref, out_ref):
    pre = (s_ref[0, :N, :] + s_ref[1, :N, :] + hp_ref[...]) * dis_ref[...][:, None]
    out_ref[...] = pre + b_ref[...][None, :]


@jax.jit
def _tc_first(deg, x, w):
    return pl.pallas_call(
        _tc_first_body,
        out_shape=(jax.ShapeDtypeStruct((N,), jnp.float32),
                   jax.ShapeDtypeStruct((N, D), jnp.float32)),
    )(deg, x, w)


@jax.jit
def _tc_mid(s_part, hp, dis, g, be, w):
    return pl.pallas_call(
        _tc_mid_body,
        out_shape=jax.ShapeDtypeStruct((N, D), jnp.float32),
    )(s_part, hp, dis, g, be, w)


@jax.jit
def _tc_final(s_part, hp, dis, b):
    return pl.pallas_call(
        _tc_final_body,
        out_shape=jax.ShapeDtypeStruct((N, D), jnp.float32),
    )(s_part, hp, dis, b)


def kernel(x, edge_index, W1, b1, g1, be1, W2, b2, g2, be2, W3, b3):
    pad = EPAD - E
    rows = jnp.concatenate(
        [edge_index[0], jnp.full((pad,), N, jnp.int32)]).reshape(NW, NCHUNK, K)
    cols = jnp.concatenate(
        [edge_index[1], jnp.full((pad,), N, jnp.int32)]).reshape(NW, NCHUNK, K)
    zeros128 = jnp.zeros((NPAD, D), jnp.float32)
    ones128 = jnp.ones((K, D), jnp.float32)

    deg = _hist(cols, zeros128, ones128)
    dis, h1p = _tc_first(deg, x, W1)
    s1 = _spmm(jnp.pad(h1p, ((0, NPAD - N), (0, 0))), rows, cols, zeros128)
    h2p = _tc_mid(s1, h1p, dis, g1, be1, W2)
    s2 = _spmm(jnp.pad(h2p, ((0, NPAD - N), (0, 0))), rows, cols, zeros128)
    h3p = _tc_mid(s2, h2p, dis, g2, be2, W3)
    s3 = _spmm(jnp.pad(h3p, ((0, NPAD - N), (0, 0))), rows, cols, zeros128)
    return _tc_final(s3, h3p, dis, b3)
